# baseline (speedup 1.0000x reference)
import functools

import jax
import jax.numpy as jnp
from jax import lax
from jax.experimental import pallas as pl
from jax.experimental.pallas import tpu as pltpu
from jax.experimental.pallas import tpu_sc as plsc

N_PLAYER = 10000
HID = 64
NROWS = 10240
STRIPE = NROWS // 16
CHUNK = 128
ZROWS = 64

_PREC = lax.Precision.HIGHEST


def _dotbf(a, b):
    return jnp.dot(a.astype(jnp.bfloat16), b.astype(jnp.bfloat16),
                   preferred_element_type=jnp.float32)


def _make_seg_sum(W: int, ch: int):
    mesh = plsc.VectorSubcoreMesh(core_axis_name="c", subcore_axis_name="s")

    if ch % 2 != 0:
        raise ValueError("chunk count must be even for the 2-buffer pipeline")

    def body(src_hbm, dst_hbm, table_hbm, out_hbm, sidx, didx, rows_a, rows_b,
             zbuf, acc, gs_a, gs_b, ss_a, ss_b):
        c = lax.axis_index("c")
        s = lax.axis_index("s")

        def zrow(i, carry):
            for j in range(W // 16):
                zbuf[i, pl.ds(j * 16, 16)] = jnp.zeros((16,), jnp.float32)
            return carry

        lax.fori_loop(0, ZROWS, zrow, 0)
        for k in range(STRIPE // ZROWS):
            pltpu.sync_copy(zbuf, acc.at[pl.ds(s * STRIPE + k * ZROWS, ZROWS)])
        plsc.subcore_barrier()

        pltpu.sync_copy(src_hbm.at[c, s], sidx)
        pltpu.sync_copy(dst_hbm.at[c, s], didx)

        def gather(i, buf, sem):
            return pltpu.make_async_copy(table_hbm.at[sidx.at[i]], buf, sem)

        def scat(i, buf, sem):
            return pltpu.make_async_copy(buf, acc.at[didx.at[i]], sem)

        pltpu.async_copy(table_hbm.at[sidx.at[0]], rows_a, gs_a)
        gather(0, rows_a, gs_a).wait()
        pltpu.async_copy(rows_a, acc.at[didx.at[0]], ss_a, add=True)
        pltpu.async_copy(table_hbm.at[sidx.at[1]], rows_b, gs_b)

        def step(k, carry):
            i1 = 2 * k - 1
            i2 = 2 * k
            gather(i1, rows_b, gs_b).wait()
            pltpu.async_copy(rows_b, acc.at[didx.at[i1]], ss_b, add=True)
            scat(i2, rows_a, ss_a).wait()
            pltpu.async_copy(table_hbm.at[sidx.at[i2]], rows_a, gs_a)
            gather(i2, rows_a, gs_a).wait()
            pltpu.async_copy(rows_a, acc.at[didx.at[i2]], ss_a, add=True)
            scat(i1, rows_b, ss_b).wait()

            @pl.when(i2 + 1 < ch)
            def _():
                pltpu.async_copy(table_hbm.at[sidx.at[i2 + 1]], rows_b, gs_b)

            return carry

        lax.fori_loop(1, ch // 2, step, 0)
        gather(ch - 1, rows_b, gs_b).wait()
        pltpu.async_copy(rows_b, acc.at[didx.at[ch - 1]], ss_b, add=True)
        scat(0, rows_a, ss_a).wait()
        scat(0, rows_b, ss_b).wait()
        plsc.subcore_barrier()

        pltpu.sync_copy(acc.at[pl.ds(s * STRIPE, STRIPE)],
                        out_hbm.at[c, pl.ds(s * STRIPE, STRIPE)])

    return pl.kernel(
        body,
        out_type=jax.ShapeDtypeStruct((2, NROWS, W), jnp.float32),
        mesh=mesh,
        compiler_params=pltpu.CompilerParams(use_tc_tiling_on_sc=False),
        scratch_types=[
            pltpu.VMEM((ch, CHUNK), jnp.int32),
            pltpu.VMEM((ch, CHUNK), jnp.int32),
            pltpu.VMEM((CHUNK, W), jnp.float32),
            pltpu.VMEM((CHUNK, W), jnp.float32),
            pltpu.VMEM((ZROWS, W), jnp.float32),
            pltpu.VMEM_SHARED((NROWS, W), jnp.float32),
            pltpu.SemaphoreType.DMA,
            pltpu.SemaphoreType.DMA,
            pltpu.SemaphoreType.DMA,
            pltpu.SemaphoreType.DMA,
        ],
    )


def _make_seg_sum4(W: int, ch: int):
    mesh = plsc.VectorSubcoreMesh(core_axis_name="c", subcore_axis_name="s")

    if ch % 4 != 0 or ch < 12:
        raise ValueError("chunk count must be a multiple of 4 (>= 12)")

    def body(src_hbm, dst_hbm, table_hbm, out_hbm, sidx, didx, rows0, rows1,
             rows2, rows3, zbuf, acc, gs0, gs1, gs2, gs3, ss0, ss1, ss2, ss3):
        c = lax.axis_index("c")
        s = lax.axis_index("s")
        rows = [rows0, rows1, rows2, rows3]
        gs = [gs0, gs1, gs2, gs3]
        ss = [ss0, ss1, ss2, ss3]

        def zrow(i, carry):
            for j in range(W // 16):
                zbuf[i, pl.ds(j * 16, 16)] = jnp.zeros((16,), jnp.float32)
            return carry

        lax.fori_loop(0, ZROWS, zrow, 0)
        for k in range(STRIPE // ZROWS):
            pltpu.sync_copy(zbuf, acc.at[pl.ds(s * STRIPE + k * ZROWS, ZROWS)])
        plsc.subcore_barrier()

        pltpu.sync_copy(src_hbm.at[c, s], sidx)
        pltpu.sync_copy(dst_hbm.at[c, s], didx)

        def g_start(i, b):
            pltpu.async_copy(table_hbm.at[sidx.at[i]], rows[b], gs[b])

        def g_wait(i, b):
            pltpu.make_async_copy(table_hbm.at[sidx.at[i]], rows[b], gs[b]).wait()

        def s_start(i, b):
            pltpu.async_copy(rows[b], acc.at[didx.at[i]], ss[b], add=True)

        def s_wait(i, b):
            pltpu.make_async_copy(rows[b], acc.at[didx.at[i]], ss[b]).wait()

        g_start(0, 0)
        g_start(1, 1)
        for r in range(4):
            if r >= 2:
                s_wait(r - 2, r - 2)
            g_start(r + 2, (r + 2) % 4)
            g_wait(r, r)
            s_start(r, r)

        def step(k, carry):
            for r in range(4):
                i = 4 * k + r
                b2 = (r + 2) % 4
                s_wait(i - 2, b2)
                g_start(i + 2, b2)
                g_wait(i, r)
                s_start(i, r)
            return carry

        lax.fori_loop(1, ch // 4 - 1, step, 0)
        for r in range(4):
            i = ch - 4 + r
            b2 = (r + 2) % 4
            s_wait(i - 2, b2)
            if i + 2 < ch:
                g_start(i + 2, b2)
            g_wait(i, r)
            s_start(i, r)
        s_wait(ch - 2, 2)
        s_wait(ch - 1, 3)
        plsc.subcore_barrier()

        pltpu.sync_copy(acc.at[pl.ds(s * STRIPE, STRIPE)],
                        out_hbm.at[c, pl.ds(s * STRIPE, STRIPE)])

    return pl.kernel(
        body,
        out_type=jax.ShapeDtypeStruct((2, NROWS, W), jnp.float32),
        mesh=mesh,
        compiler_params=pltpu.CompilerParams(use_tc_tiling_on_sc=False),
        scratch_types=[
            pltpu.VMEM((ch, CHUNK), jnp.int32),
            pltpu.VMEM((ch, CHUNK), jnp.int32),
            pltpu.VMEM((CHUNK, W), jnp.float32),
            pltpu.VMEM((CHUNK, W), jnp.float32),
            pltpu.VMEM((CHUNK, W), jnp.float32),
            pltpu.VMEM((CHUNK, W), jnp.float32),
            pltpu.VMEM((ZROWS, W), jnp.float32),
            pltpu.VMEM_SHARED((NROWS, W), jnp.float32),
            pltpu.SemaphoreType.DMA,
            pltpu.SemaphoreType.DMA,
            pltpu.SemaphoreType.DMA,
            pltpu.SemaphoreType.DMA,
            pltpu.SemaphoreType.DMA,
            pltpu.SemaphoreType.DMA,
            pltpu.SemaphoreType.DMA,
            pltpu.SemaphoreType.DMA,
        ],
    )


def _prep_edges(ei):
    src = ei[0].astype(jnp.int32)
    dst = ei[1].astype(jnp.int32)
    ne = src.shape[0]
    ept = -(-ne // (16 * 4 * CHUNK)) * 4 * CHUNK
    pad = 16 * ept - ne
    src = jnp.concatenate([src, jnp.zeros((pad,), jnp.int32)])
    dst = jnp.concatenate([dst, jnp.full((pad,), N_PLAYER, jnp.int32)])
    return src.reshape(16, ept // CHUNK, CHUNK), dst.reshape(16, ept // CHUNK, CHUNK)


_BLK = 1000


def _enc_body(x_ref, w_ref, b_ref, o_ref):
    o_ref[...] = _dotbf(x_ref[...], w_ref[...]) + b_ref[...]


def _encode(x, w_aug, b_aug):
    m, k = x.shape
    w = w_aug.shape[1]
    return pl.pallas_call(
        _enc_body,
        grid=(m // _BLK,),
        in_specs=[
            pl.BlockSpec((_BLK, k), lambda i: (i, 0)),
            pl.BlockSpec((k, w), lambda i: (0, 0)),
            pl.BlockSpec((1, w), lambda i: (0, 0)),
        ],
        out_specs=pl.BlockSpec((_BLK, w), lambda i: (i, 0)),
        out_shape=jax.ShapeDtypeStruct((m, w), jnp.float32),
    )(x, w_aug, b_aug)


def _comb_body(sf_ref, cf_ref, sb_ref, cb_ref, h_ref, wlf_ref, blf_ref,
               wrf_ref, wlb_ref, blb_ref, wrb_ref, o_ref, ps_ref):
    af = sf_ref[...] / jnp.maximum(cf_ref[...], 1.0)
    ab = sb_ref[...] / jnp.maximum(cb_ref[...], 1.0)
    h = h_ref[...]
    out_f = _dotbf(af, wlf_ref[...]) + blf_ref[...] + _dotbf(h, wrf_ref[...])
    out_b = _dotbf(ab, wlb_ref[...]) + blb_ref[...] + _dotbf(h, wrb_ref[...])
    hn = jnp.maximum((out_f + out_b) * 0.5, 0.0)
    o_ref[...] = hn

    @pl.when(pl.program_id(0) == 0)
    def _():
        ps_ref[...] = jnp.zeros_like(ps_ref)

    ps_ref[...] += jnp.sum(hn, axis=0, keepdims=True)


def _combine(sf, cf, sb, cb, h, wlf, blf, wrf, wlb, blb, wrb):
    m = h.shape[0]
    return pl.pallas_call(
        _comb_body,
        grid=(m // _BLK,),
        in_specs=[
            pl.BlockSpec((_BLK, HID), lambda i: (i, 0)),
            pl.BlockSpec((_BLK, 1), lambda i: (i, 0)),
            pl.BlockSpec((_BLK, HID), lambda i: (i, 0)),
            pl.BlockSpec((_BLK, 1), lambda i: (i, 0)),
            pl.BlockSpec((_BLK, HID), lambda i: (i, 0)),
            pl.BlockSpec((HID, HID), lambda i: (0, 0)),
            pl.BlockSpec((1, HID), lambda i: (0, 0)),
            pl.BlockSpec((HID, HID), lambda i: (0, 0)),
            pl.BlockSpec((HID, HID), lambda i: (0, 0)),
            pl.BlockSpec((1, HID), lambda i: (0, 0)),
            pl.BlockSpec((HID, HID), lambda i: (0, 0)),
        ],
        out_specs=[
            pl.BlockSpec((_BLK, HID), lambda i: (i, 0)),
            pl.BlockSpec((1, HID), lambda i: (0, 0)),
        ],
        out_shape=[
            jax.ShapeDtypeStruct((m, HID), jnp.float32),
            jax.ShapeDtypeStruct((1, HID), jnp.float32),
        ],
    )(sf, cf, sb, cb, h, wlf, blf, wrf, wlb, blb, wrb)


def _dot(a, b):
    return _dotbf(a, b)


def _tail_body(ps_ref, wf_ref, vf_ref, role_ref, bat_ref, bowl_ref, exp_ref,
               ww_ref, bw_ref, wve_ref, bve_ref, rolet_ref, batt_ref, bowlt_ref,
               ew_ref, eb_ref, aw_ref, ab_ref, pw_ref, pb_ref, wv_ref, bv_ref,
               wo_ref, bo_ref, w1_ref, b1_ref, w2_ref, b2_ref, w3_ref, b3_ref,
               wa_ref, ba_ref, wb2_ref, bb2_ref,
               mp_ref, wip_ref, pe_ref, att_ref, ve_ref, te_ref):
    pe = ps_ref[...] / float(N_PLAYER)
    weather_emb = _dot(wf_ref[...], ww_ref[...]) + bw_ref[...]
    ve = _dot(vf_ref[...], wve_ref[...]) + bve_ref[...]
    v = _dot(weather_emb, wv_ref[...]) + bv_ref[...]
    att = _dot(v, wo_ref[...]) + bo_ref[...]

    def onehot(idx_ref, depth):
        idx = idx_ref[...]
        io = lax.broadcasted_iota(jnp.int32, (22, depth), 1)
        return (io == idx).astype(jnp.float32)

    def exact_dot(a, b):
        return jnp.dot(a, b, preferred_element_type=jnp.float32, precision=_PREC)

    role_e = exact_dot(onehot(role_ref, 5), rolet_ref[...])
    bat_e = exact_dot(onehot(bat_ref, 3), batt_ref[...])
    bowl_e = exact_dot(onehot(bowl_ref, 9), bowlt_ref[...])
    exp_e = _dot(exp_ref[...], ew_ref[...]) + eb_ref[...]
    pemb = jnp.concatenate([role_e, bat_e, bowl_e, exp_e], axis=1)
    sm = jnp.concatenate(
        [jnp.mean(pemb[0:11, :], axis=0, keepdims=True),
         jnp.mean(pemb[11:22, :], axis=0, keepdims=True)], axis=0)
    squad_emb = _dot(sm, aw_ref[...]) + ab_ref[...]
    te = jnp.mean(_dot(squad_emb, pw_ref[...]) + pb_ref[...], axis=0,
                  keepdims=True)

    combined = jnp.concatenate([pe, att, ve, te], axis=1)
    hh = jnp.maximum(exact_dot(combined, w1_ref[...]) + b1_ref[...], 0.0)
    hh = jnp.maximum(exact_dot(hh, w2_ref[...]) + b2_ref[...], 0.0)
    mp_ref[...] = exact_dot(hh, w3_ref[...]) + b3_ref[...]
    wip_ref[...] = _dot(jnp.maximum(_dot(att, wa_ref[...]) + ba_ref[...], 0.0),
                        wb2_ref[...]) + bb2_ref[...]
    pe_ref[...] = pe
    att_ref[...] = att
    ve_ref[...] = ve
    te_ref[...] = te


def _tail(*args):
    return pl.pallas_call(
        _tail_body,
        out_shape=[
            jax.ShapeDtypeStruct((1, 1), jnp.float32),
            jax.ShapeDtypeStruct((1, 3), jnp.float32),
            jax.ShapeDtypeStruct((1, HID), jnp.float32),
            jax.ShapeDtypeStruct((1, HID), jnp.float32),
            jax.ShapeDtypeStruct((1, HID), jnp.float32),
            jax.ShapeDtypeStruct((1, HID), jnp.float32),
        ],
    )(*args)


def kernel(x_player, x_venue, x_team, x_match, x_weather, ei_faced,
           ei_bowled_to, ei_played_at_pv, ei_plays_for, ei_played_at_mv,
           ei_had_weather, ei_played_in, weather_features, venue_features,
           role_idx, bat_idx, bowl_idx, exp_feats, params):
    sF, dF = _prep_edges(ei_faced)
    sB, dB = _prep_edges(ei_bowled_to)
    src_idx = jnp.stack([sF, sB])
    dst_idx = jnp.stack([dF, dB])
    ch = src_idx.shape[2]

    We, be = params["enc"]["player"]
    W_aug = jnp.pad(We, ((0, 0), (0, 16)))
    b_aug = jnp.concatenate(
        [be, jnp.ones((1,), jnp.float32),
         jnp.zeros((15,), jnp.float32)])[None]
    h_aug = _encode(x_player, W_aug, b_aug)

    seg80 = _make_seg_sum(HID + 16, ch)
    seg64 = _make_seg_sum(HID, ch)

    acc0 = seg80(src_idx, dst_idx, h_aug)
    cF = acc0[0, :N_PLAYER, HID:HID + 1]
    cB = acc0[1, :N_PLAYER, HID:HID + 1]

    h = h_aug[:, :HID]
    acc = acc0
    ps = None
    for li, layer in enumerate(params["convs"]):
        (WlF, blF, WrF), (WlB, blB, WrB) = layer[0], layer[1]
        h, ps = _combine(
            acc[0, :N_PLAYER, :HID], cF, acc[1, :N_PLAYER, :HID], cB, h,
            WlF, blF[None], WrF, WlB, blB[None], WrB)
        if li < 2:
            acc = seg64(src_idx, dst_idx, h)

    sq = params["squad"]
    at = params["attn"]
    (W1, b1), (W2, b2), (W3, b3) = params["mp"]
    (Wa, ba), (Wb2, bb2) = params["wip"]
    mp, wip, pe, att, ve, te = _tail(
        ps, weather_features[None], venue_features[None],
        role_idx.reshape(22, 1), bat_idx.reshape(22, 1),
        bowl_idx.reshape(22, 1), exp_feats.reshape(22, 4),
        params["weather_enc"][0], params["weather_enc"][1][None],
        params["venue_enc"][0], params["venue_enc"][1][None],
        sq["role"], sq["bat"], sq["bowl"], sq["exp"][0], sq["exp"][1][None],
        sq["agg"][0], sq["agg"][1][None],
        params["proj"][0], params["proj"][1][None],
        at["Wv"][0], at["Wv"][1][None], at["Wo"][0], at["Wo"][1][None],
        W1, b1[None], W2, b2[None], W3, b3[None],
        Wa, ba[None], Wb2, bb2[None])

    return (mp.reshape(1), wip.reshape(3), pe.reshape(HID), att.reshape(HID),
            ve.reshape(HID), te.reshape(HID))


# speedup vs baseline: 2.1822x; 2.1822x over previous
import functools

import jax
import jax.numpy as jnp
from jax import lax
from jax.experimental import pallas as pl
from jax.experimental.pallas import tpu as pltpu
from jax.experimental.pallas import tpu_sc as plsc

N_PLAYER = 10000
HID = 64
NROWS = 10240
STRIPE = NROWS // 16
CHUNK = 128
ZROWS = 64

_PREC = lax.Precision.HIGHEST


def _dotbf(a, b):
    return jnp.dot(a.astype(jnp.bfloat16), b.astype(jnp.bfloat16),
                   preferred_element_type=jnp.float32)


def _make_seg_sum(W: int, ch: int):
    mesh = plsc.VectorSubcoreMesh(core_axis_name="c", subcore_axis_name="s")

    if ch % 2 != 0:
        raise ValueError("chunk count must be even for the 2-buffer pipeline")

    def body(src_hbm, dst_hbm, table_hbm, out_hbm, sidx, didx, rows_a, rows_b,
             zbuf, acc, gs_a, gs_b, ss_a, ss_b):
        c = lax.axis_index("c")
        s = lax.axis_index("s")

        def zrow(i, carry):
            for j in range(W // 16):
                zbuf[i, pl.ds(j * 16, 16)] = jnp.zeros((16,), jnp.float32)
            return carry

        lax.fori_loop(0, ZROWS, zrow, 0)
        for k in range(STRIPE // ZROWS):
            pltpu.sync_copy(zbuf, acc.at[pl.ds(s * STRIPE + k * ZROWS, ZROWS)])
        plsc.subcore_barrier()

        pltpu.sync_copy(src_hbm.at[c, s], sidx)
        pltpu.sync_copy(dst_hbm.at[c, s], didx)

        def gather(i, buf, sem):
            return pltpu.make_async_copy(table_hbm.at[sidx.at[i]], buf, sem)

        def scat(i, buf, sem):
            return pltpu.make_async_copy(buf, acc.at[didx.at[i]], sem)

        pltpu.async_copy(table_hbm.at[sidx.at[0]], rows_a, gs_a)
        gather(0, rows_a, gs_a).wait()
        pltpu.async_copy(rows_a, acc.at[didx.at[0]], ss_a, add=True)
        pltpu.async_copy(table_hbm.at[sidx.at[1]], rows_b, gs_b)

        def step(k, carry):
            i1 = 2 * k - 1
            i2 = 2 * k
            gather(i1, rows_b, gs_b).wait()
            pltpu.async_copy(rows_b, acc.at[didx.at[i1]], ss_b, add=True)
            scat(i2, rows_a, ss_a).wait()
            pltpu.async_copy(table_hbm.at[sidx.at[i2]], rows_a, gs_a)
            gather(i2, rows_a, gs_a).wait()
            pltpu.async_copy(rows_a, acc.at[didx.at[i2]], ss_a, add=True)
            scat(i1, rows_b, ss_b).wait()

            @pl.when(i2 + 1 < ch)
            def _():
                pltpu.async_copy(table_hbm.at[sidx.at[i2 + 1]], rows_b, gs_b)

            return carry

        lax.fori_loop(1, ch // 2, step, 0)
        gather(ch - 1, rows_b, gs_b).wait()
        pltpu.async_copy(rows_b, acc.at[didx.at[ch - 1]], ss_b, add=True)
        scat(0, rows_a, ss_a).wait()
        scat(0, rows_b, ss_b).wait()
        plsc.subcore_barrier()

        pltpu.sync_copy(acc.at[pl.ds(s * STRIPE, STRIPE)],
                        out_hbm.at[c, pl.ds(s * STRIPE, STRIPE)])

    return pl.kernel(
        body,
        out_type=jax.ShapeDtypeStruct((2, NROWS, W), jnp.float32),
        mesh=mesh,
        compiler_params=pltpu.CompilerParams(use_tc_tiling_on_sc=False),
        scratch_types=[
            pltpu.VMEM((ch, CHUNK), jnp.int32),
            pltpu.VMEM((ch, CHUNK), jnp.int32),
            pltpu.VMEM((CHUNK, W), jnp.float32),
            pltpu.VMEM((CHUNK, W), jnp.float32),
            pltpu.VMEM((ZROWS, W), jnp.float32),
            pltpu.VMEM_SHARED((NROWS, W), jnp.float32),
            pltpu.SemaphoreType.DMA,
            pltpu.SemaphoreType.DMA,
            pltpu.SemaphoreType.DMA,
            pltpu.SemaphoreType.DMA,
        ],
    )


def _make_seg_sum4(W: int, ch: int):
    mesh = plsc.VectorSubcoreMesh(core_axis_name="c", subcore_axis_name="s")

    if ch % 4 != 0 or ch < 12:
        raise ValueError("chunk count must be a multiple of 4 (>= 12)")

    def body(src_hbm, dst_hbm, table_hbm, out_hbm, sidx, didx, rows0, rows1,
             rows2, rows3, zbuf, acc, gs0, gs1, gs2, gs3, ss0, ss1, ss2, ss3):
        c = lax.axis_index("c")
        s = lax.axis_index("s")
        rows = [rows0, rows1, rows2, rows3]
        gs = [gs0, gs1, gs2, gs3]
        ss = [ss0, ss1, ss2, ss3]

        def zrow(i, carry):
            for j in range(W // 16):
                zbuf[i, pl.ds(j * 16, 16)] = jnp.zeros((16,), jnp.float32)
            return carry

        lax.fori_loop(0, ZROWS, zrow, 0)
        for k in range(STRIPE // ZROWS):
            pltpu.sync_copy(zbuf, acc.at[pl.ds(s * STRIPE + k * ZROWS, ZROWS)])
        plsc.subcore_barrier()

        pltpu.sync_copy(src_hbm.at[c, s], sidx)
        pltpu.sync_copy(dst_hbm.at[c, s], didx)

        def g_start(i, b):
            pltpu.async_copy(table_hbm.at[sidx.at[i]], rows[b], gs[b])

        def g_wait(i, b):
            pltpu.make_async_copy(table_hbm.at[sidx.at[i]], rows[b], gs[b]).wait()

        def s_start(i, b):
            pltpu.async_copy(rows[b], acc.at[didx.at[i]], ss[b], add=True)

        def s_wait(i, b):
            pltpu.make_async_copy(rows[b], acc.at[didx.at[i]], ss[b]).wait()

        g_start(0, 0)
        g_start(1, 1)
        for r in range(4):
            if r >= 2:
                s_wait(r - 2, r - 2)
            g_start(r + 2, (r + 2) % 4)
            g_wait(r, r)
            s_start(r, r)

        def step(k, carry):
            for r in range(4):
                i = 4 * k + r
                b2 = (r + 2) % 4
                s_wait(i - 2, b2)
                g_start(i + 2, b2)
                g_wait(i, r)
                s_start(i, r)
            return carry

        lax.fori_loop(1, ch // 4 - 1, step, 0)
        for r in range(4):
            i = ch - 4 + r
            b2 = (r + 2) % 4
            s_wait(i - 2, b2)
            if i + 2 < ch:
                g_start(i + 2, b2)
            g_wait(i, r)
            s_start(i, r)
        s_wait(ch - 2, 2)
        s_wait(ch - 1, 3)
        plsc.subcore_barrier()

        pltpu.sync_copy(acc.at[pl.ds(s * STRIPE, STRIPE)],
                        out_hbm.at[c, pl.ds(s * STRIPE, STRIPE)])

    return pl.kernel(
        body,
        out_type=jax.ShapeDtypeStruct((2, NROWS, W), jnp.float32),
        mesh=mesh,
        compiler_params=pltpu.CompilerParams(use_tc_tiling_on_sc=False),
        scratch_types=[
            pltpu.VMEM((ch, CHUNK), jnp.int32),
            pltpu.VMEM((ch, CHUNK), jnp.int32),
            pltpu.VMEM((CHUNK, W), jnp.float32),
            pltpu.VMEM((CHUNK, W), jnp.float32),
            pltpu.VMEM((CHUNK, W), jnp.float32),
            pltpu.VMEM((CHUNK, W), jnp.float32),
            pltpu.VMEM((ZROWS, W), jnp.float32),
            pltpu.VMEM_SHARED((NROWS, W), jnp.float32),
            pltpu.SemaphoreType.DMA,
            pltpu.SemaphoreType.DMA,
            pltpu.SemaphoreType.DMA,
            pltpu.SemaphoreType.DMA,
            pltpu.SemaphoreType.DMA,
            pltpu.SemaphoreType.DMA,
            pltpu.SemaphoreType.DMA,
            pltpu.SemaphoreType.DMA,
        ],
    )


def _prep_edges(ei):
    src = ei[0].astype(jnp.int32)
    dst = ei[1].astype(jnp.int32)
    ne = src.shape[0]
    ept = -(-ne // (16 * 4 * CHUNK)) * 4 * CHUNK
    pad = 16 * ept - ne
    iota = jnp.arange(pad, dtype=jnp.int32)
    src = jnp.concatenate([src, iota % N_PLAYER])
    dst = jnp.concatenate([dst, N_PLAYER + (iota % (NROWS - N_PLAYER))])
    return src.reshape(16, ept // CHUNK, CHUNK), dst.reshape(16, ept // CHUNK, CHUNK)


_BLK = 1000


def _enc_body(x_ref, w_ref, b_ref, o_ref):
    o_ref[...] = _dotbf(x_ref[...], w_ref[...]) + b_ref[...]


def _encode(x, w_aug, b_aug):
    m, k = x.shape
    w = w_aug.shape[1]
    return pl.pallas_call(
        _enc_body,
        grid=(m // _BLK,),
        in_specs=[
            pl.BlockSpec((_BLK, k), lambda i: (i, 0)),
            pl.BlockSpec((k, w), lambda i: (0, 0)),
            pl.BlockSpec((1, w), lambda i: (0, 0)),
        ],
        out_specs=pl.BlockSpec((_BLK, w), lambda i: (i, 0)),
        out_shape=jax.ShapeDtypeStruct((m, w), jnp.float32),
    )(x, w_aug, b_aug)


def _comb_body(sf_ref, cf_ref, sb_ref, cb_ref, h_ref, wlf_ref, blf_ref,
               wrf_ref, wlb_ref, blb_ref, wrb_ref, o_ref, ps_ref):
    af = sf_ref[...] / jnp.maximum(cf_ref[...], 1.0)
    ab = sb_ref[...] / jnp.maximum(cb_ref[...], 1.0)
    h = h_ref[...]
    out_f = _dotbf(af, wlf_ref[...]) + blf_ref[...] + _dotbf(h, wrf_ref[...])
    out_b = _dotbf(ab, wlb_ref[...]) + blb_ref[...] + _dotbf(h, wrb_ref[...])
    hn = jnp.maximum((out_f + out_b) * 0.5, 0.0)
    o_ref[...] = hn

    @pl.when(pl.program_id(0) == 0)
    def _():
        ps_ref[...] = jnp.zeros_like(ps_ref)

    ps_ref[...] += jnp.sum(hn, axis=0, keepdims=True)


def _combine(sf, cf, sb, cb, h, wlf, blf, wrf, wlb, blb, wrb):
    m = h.shape[0]
    return pl.pallas_call(
        _comb_body,
        grid=(m // _BLK,),
        in_specs=[
            pl.BlockSpec((_BLK, HID), lambda i: (i, 0)),
            pl.BlockSpec((_BLK, 1), lambda i: (i, 0)),
            pl.BlockSpec((_BLK, HID), lambda i: (i, 0)),
            pl.BlockSpec((_BLK, 1), lambda i: (i, 0)),
            pl.BlockSpec((_BLK, HID), lambda i: (i, 0)),
            pl.BlockSpec((HID, HID), lambda i: (0, 0)),
            pl.BlockSpec((1, HID), lambda i: (0, 0)),
            pl.BlockSpec((HID, HID), lambda i: (0, 0)),
            pl.BlockSpec((HID, HID), lambda i: (0, 0)),
            pl.BlockSpec((1, HID), lambda i: (0, 0)),
            pl.BlockSpec((HID, HID), lambda i: (0, 0)),
        ],
        out_specs=[
            pl.BlockSpec((_BLK, HID), lambda i: (i, 0)),
            pl.BlockSpec((1, HID), lambda i: (0, 0)),
        ],
        out_shape=[
            jax.ShapeDtypeStruct((m, HID), jnp.float32),
            jax.ShapeDtypeStruct((1, HID), jnp.float32),
        ],
    )(sf, cf, sb, cb, h, wlf, blf, wrf, wlb, blb, wrb)


def _dot(a, b):
    return _dotbf(a, b)


def _tail_body(ps_ref, wf_ref, vf_ref, role_ref, bat_ref, bowl_ref, exp_ref,
               ww_ref, bw_ref, wve_ref, bve_ref, rolet_ref, batt_ref, bowlt_ref,
               ew_ref, eb_ref, aw_ref, ab_ref, pw_ref, pb_ref, wv_ref, bv_ref,
               wo_ref, bo_ref, w1_ref, b1_ref, w2_ref, b2_ref, w3_ref, b3_ref,
               wa_ref, ba_ref, wb2_ref, bb2_ref,
               mp_ref, wip_ref, pe_ref, att_ref, ve_ref, te_ref):
    pe = ps_ref[...] / float(N_PLAYER)
    weather_emb = _dot(wf_ref[...], ww_ref[...]) + bw_ref[...]
    ve = _dot(vf_ref[...], wve_ref[...]) + bve_ref[...]
    v = _dot(weather_emb, wv_ref[...]) + bv_ref[...]
    att = _dot(v, wo_ref[...]) + bo_ref[...]

    def onehot(idx_ref, depth):
        idx = idx_ref[...]
        io = lax.broadcasted_iota(jnp.int32, (22, depth), 1)
        return (io == idx).astype(jnp.float32)

    def exact_dot(a, b):
        return jnp.dot(a, b, preferred_element_type=jnp.float32, precision=_PREC)

    role_e = exact_dot(onehot(role_ref, 5), rolet_ref[...])
    bat_e = exact_dot(onehot(bat_ref, 3), batt_ref[...])
    bowl_e = exact_dot(onehot(bowl_ref, 9), bowlt_ref[...])
    exp_e = _dot(exp_ref[...], ew_ref[...]) + eb_ref[...]
    pemb = jnp.concatenate([role_e, bat_e, bowl_e, exp_e], axis=1)
    sm = jnp.concatenate(
        [jnp.mean(pemb[0:11, :], axis=0, keepdims=True),
         jnp.mean(pemb[11:22, :], axis=0, keepdims=True)], axis=0)
    squad_emb = _dot(sm, aw_ref[...]) + ab_ref[...]
    te = jnp.mean(_dot(squad_emb, pw_ref[...]) + pb_ref[...], axis=0,
                  keepdims=True)

    combined = jnp.concatenate([pe, att, ve, te], axis=1)
    hh = jnp.maximum(exact_dot(combined, w1_ref[...]) + b1_ref[...], 0.0)
    hh = jnp.maximum(exact_dot(hh, w2_ref[...]) + b2_ref[...], 0.0)
    mp_ref[...] = exact_dot(hh, w3_ref[...]) + b3_ref[...]
    wip_ref[...] = _dot(jnp.maximum(_dot(att, wa_ref[...]) + ba_ref[...], 0.0),
                        wb2_ref[...]) + bb2_ref[...]
    pe_ref[...] = pe
    att_ref[...] = att
    ve_ref[...] = ve
    te_ref[...] = te


def _tail(*args):
    return pl.pallas_call(
        _tail_body,
        out_shape=[
            jax.ShapeDtypeStruct((1, 1), jnp.float32),
            jax.ShapeDtypeStruct((1, 3), jnp.float32),
            jax.ShapeDtypeStruct((1, HID), jnp.float32),
            jax.ShapeDtypeStruct((1, HID), jnp.float32),
            jax.ShapeDtypeStruct((1, HID), jnp.float32),
            jax.ShapeDtypeStruct((1, HID), jnp.float32),
        ],
    )(*args)


def kernel(x_player, x_venue, x_team, x_match, x_weather, ei_faced,
           ei_bowled_to, ei_played_at_pv, ei_plays_for, ei_played_at_mv,
           ei_had_weather, ei_played_in, weather_features, venue_features,
           role_idx, bat_idx, bowl_idx, exp_feats, params):
    sF, dF = _prep_edges(ei_faced)
    sB, dB = _prep_edges(ei_bowled_to)
    src_idx = jnp.stack([sF, sB])
    dst_idx = jnp.stack([dF, dB])
    ch = src_idx.shape[2]

    We, be = params["enc"]["player"]
    W_aug = jnp.pad(We, ((0, 0), (0, 16)))
    b_aug = jnp.concatenate(
        [be, jnp.ones((1,), jnp.float32),
         jnp.zeros((15,), jnp.float32)])[None]
    h_aug = _encode(x_player, W_aug, b_aug)

    seg80 = _make_seg_sum(HID + 16, ch)
    seg64 = _make_seg_sum(HID, ch)

    acc0 = seg80(src_idx, dst_idx, h_aug)
    cF = acc0[0, :N_PLAYER, HID:HID + 1]
    cB = acc0[1, :N_PLAYER, HID:HID + 1]

    h = h_aug[:, :HID]
    acc = acc0
    ps = None
    for li, layer in enumerate(params["convs"]):
        (WlF, blF, WrF), (WlB, blB, WrB) = layer[0], layer[1]
        h, ps = _combine(
            acc[0, :N_PLAYER, :HID], cF, acc[1, :N_PLAYER, :HID], cB, h,
            WlF, blF[None], WrF, WlB, blB[None], WrB)
        if li < 2:
            acc = seg64(src_idx, dst_idx, h)

    sq = params["squad"]
    at = params["attn"]
    (W1, b1), (W2, b2), (W3, b3) = params["mp"]
    (Wa, ba), (Wb2, bb2) = params["wip"]
    mp, wip, pe, att, ve, te = _tail(
        ps, weather_features[None], venue_features[None],
        role_idx.reshape(22, 1), bat_idx.reshape(22, 1),
        bowl_idx.reshape(22, 1), exp_feats.reshape(22, 4),
        params["weather_enc"][0], params["weather_enc"][1][None],
        params["venue_enc"][0], params["venue_enc"][1][None],
        sq["role"], sq["bat"], sq["bowl"], sq["exp"][0], sq["exp"][1][None],
        sq["agg"][0], sq["agg"][1][None],
        params["proj"][0], params["proj"][1][None],
        at["Wv"][0], at["Wv"][1][None], at["Wo"][0], at["Wo"][1][None],
        W1, b1[None], W2, b2[None], W3, b3[None],
        Wa, ba[None], Wb2, bb2[None])

    return (mp.reshape(1), wip.reshape(3), pe.reshape(HID), att.reshape(HID),
            ve.reshape(HID), te.reshape(HID))


# speedup vs baseline: 2.6872x; 1.2314x over previous
import functools

import jax
import jax.numpy as jnp
from jax import lax
from jax.experimental import pallas as pl
from jax.experimental.pallas import tpu as pltpu
from jax.experimental.pallas import tpu_sc as plsc

N_PLAYER = 10000
HID = 64
NROWS = 10240
STRIPE = NROWS // 16
CHUNK = 128
ZROWS = 64

_PREC = lax.Precision.HIGHEST


def _dotbf(a, b):
    return jnp.dot(a.astype(jnp.bfloat16), b.astype(jnp.bfloat16),
                   preferred_element_type=jnp.float32)


def _make_seg_sum(W: int, ch: int):
    mesh = plsc.VectorSubcoreMesh(core_axis_name="c", subcore_axis_name="s")

    if ch % 2 != 0:
        raise ValueError("chunk count must be even for the 2-buffer pipeline")

    def body(src_hbm, dst_hbm, table_hbm, out_hbm, sidx, didx, rows_a, rows_b,
             zbuf, acc, gs_a, gs_b, ss_a, ss_b):
        c = lax.axis_index("c")
        s = lax.axis_index("s")

        def zrow(i, carry):
            for j in range(W // 16):
                zbuf[i, pl.ds(j * 16, 16)] = jnp.zeros((16,), jnp.float32)
            return carry

        lax.fori_loop(0, ZROWS, zrow, 0)
        for k in range(STRIPE // ZROWS):
            pltpu.sync_copy(zbuf, acc.at[pl.ds(s * STRIPE + k * ZROWS, ZROWS)])
        plsc.subcore_barrier()

        pltpu.sync_copy(src_hbm.at[c, s], sidx)
        pltpu.sync_copy(dst_hbm.at[c, s], didx)

        def gather(i, buf, sem):
            return pltpu.make_async_copy(table_hbm.at[sidx.at[i]], buf, sem)

        def scat(i, buf, sem):
            return pltpu.make_async_copy(buf, acc.at[didx.at[i]], sem)

        pltpu.async_copy(table_hbm.at[sidx.at[0]], rows_a, gs_a)
        gather(0, rows_a, gs_a).wait()
        pltpu.async_copy(rows_a, acc.at[didx.at[0]], ss_a, add=True)
        pltpu.async_copy(table_hbm.at[sidx.at[1]], rows_b, gs_b)

        def step(k, carry):
            i1 = 2 * k - 1
            i2 = 2 * k
            gather(i1, rows_b, gs_b).wait()
            pltpu.async_copy(rows_b, acc.at[didx.at[i1]], ss_b, add=True)
            scat(i2, rows_a, ss_a).wait()
            pltpu.async_copy(table_hbm.at[sidx.at[i2]], rows_a, gs_a)
            gather(i2, rows_a, gs_a).wait()
            pltpu.async_copy(rows_a, acc.at[didx.at[i2]], ss_a, add=True)
            scat(i1, rows_b, ss_b).wait()

            @pl.when(i2 + 1 < ch)
            def _():
                pltpu.async_copy(table_hbm.at[sidx.at[i2 + 1]], rows_b, gs_b)

            return carry

        lax.fori_loop(1, ch // 2, step, 0)
        gather(ch - 1, rows_b, gs_b).wait()
        pltpu.async_copy(rows_b, acc.at[didx.at[ch - 1]], ss_b, add=True)
        scat(0, rows_a, ss_a).wait()
        scat(0, rows_b, ss_b).wait()
        plsc.subcore_barrier()

        pltpu.sync_copy(acc.at[pl.ds(s * STRIPE, STRIPE)],
                        out_hbm.at[c, pl.ds(s * STRIPE, STRIPE)])

    return pl.kernel(
        body,
        out_type=jax.ShapeDtypeStruct((2, NROWS, W), jnp.float32),
        mesh=mesh,
        compiler_params=pltpu.CompilerParams(use_tc_tiling_on_sc=False),
        scratch_types=[
            pltpu.VMEM((ch, CHUNK), jnp.int32),
            pltpu.VMEM((ch, CHUNK), jnp.int32),
            pltpu.VMEM((CHUNK, W), jnp.float32),
            pltpu.VMEM((CHUNK, W), jnp.float32),
            pltpu.VMEM((ZROWS, W), jnp.float32),
            pltpu.VMEM_SHARED((NROWS, W), jnp.float32),
            pltpu.SemaphoreType.DMA,
            pltpu.SemaphoreType.DMA,
            pltpu.SemaphoreType.DMA,
            pltpu.SemaphoreType.DMA,
        ],
    )


def _make_seg_sum4(W: int, ch: int):
    mesh = plsc.VectorSubcoreMesh(core_axis_name="c", subcore_axis_name="s")

    if ch % 4 != 0 or ch < 12:
        raise ValueError("chunk count must be a multiple of 4 (>= 12)")

    def body(src_hbm, dst_hbm, table_hbm, out_hbm, sidx, didx, rows0, rows1,
             rows2, rows3, zbuf, acc, gs0, gs1, gs2, gs3, ss0, ss1, ss2, ss3):
        c = lax.axis_index("c")
        s = lax.axis_index("s")
        rows = [rows0, rows1, rows2, rows3]
        gs = [gs0, gs1, gs2, gs3]
        ss = [ss0, ss1, ss2, ss3]

        def zrow(i, carry):
            for j in range(W // 16):
                zbuf[i, pl.ds(j * 16, 16)] = jnp.zeros((16,), jnp.float32)
            return carry

        lax.fori_loop(0, ZROWS, zrow, 0)
        for k in range(STRIPE // ZROWS):
            pltpu.sync_copy(zbuf, acc.at[pl.ds(s * STRIPE + k * ZROWS, ZROWS)])
        plsc.subcore_barrier()

        pltpu.sync_copy(src_hbm.at[c, s], sidx)
        pltpu.sync_copy(dst_hbm.at[c, s], didx)

        def g_start(i, b):
            pltpu.async_copy(table_hbm.at[sidx.at[i]], rows[b], gs[b])

        def g_wait(i, b):
            pltpu.make_async_copy(table_hbm.at[sidx.at[i]], rows[b], gs[b]).wait()

        def s_start(i, b):
            pltpu.async_copy(rows[b], acc.at[didx.at[i]], ss[b], add=True)

        def s_wait(i, b):
            pltpu.make_async_copy(rows[b], acc.at[didx.at[i]], ss[b]).wait()

        g_start(0, 0)
        g_start(1, 1)
        for r in range(4):
            if r >= 2:
                s_wait(r - 2, r - 2)
            g_start(r + 2, (r + 2) % 4)
            g_wait(r, r)
            s_start(r, r)

        def step(k, carry):
            for r in range(4):
                i = 4 * k + r
                b2 = (r + 2) % 4
                s_wait(i - 2, b2)
                g_start(i + 2, b2)
                g_wait(i, r)
                s_start(i, r)
            return carry

        lax.fori_loop(1, ch // 4 - 1, step, 0)
        for r in range(4):
            i = ch - 4 + r
            b2 = (r + 2) % 4
            s_wait(i - 2, b2)
            if i + 2 < ch:
                g_start(i + 2, b2)
            g_wait(i, r)
            s_start(i, r)
        s_wait(ch - 2, 2)
        s_wait(ch - 1, 3)
        plsc.subcore_barrier()

        pltpu.sync_copy(acc.at[pl.ds(s * STRIPE, STRIPE)],
                        out_hbm.at[c, pl.ds(s * STRIPE, STRIPE)])

    return pl.kernel(
        body,
        out_type=jax.ShapeDtypeStruct((2, NROWS, W), jnp.float32),
        mesh=mesh,
        compiler_params=pltpu.CompilerParams(use_tc_tiling_on_sc=False),
        scratch_types=[
            pltpu.VMEM((ch, CHUNK), jnp.int32),
            pltpu.VMEM((ch, CHUNK), jnp.int32),
            pltpu.VMEM((CHUNK, W), jnp.float32),
            pltpu.VMEM((CHUNK, W), jnp.float32),
            pltpu.VMEM((CHUNK, W), jnp.float32),
            pltpu.VMEM((CHUNK, W), jnp.float32),
            pltpu.VMEM((ZROWS, W), jnp.float32),
            pltpu.VMEM_SHARED((NROWS, W), jnp.float32),
            pltpu.SemaphoreType.DMA,
            pltpu.SemaphoreType.DMA,
            pltpu.SemaphoreType.DMA,
            pltpu.SemaphoreType.DMA,
            pltpu.SemaphoreType.DMA,
            pltpu.SemaphoreType.DMA,
            pltpu.SemaphoreType.DMA,
            pltpu.SemaphoreType.DMA,
        ],
    )


def _prep_edges(ei):
    src = ei[0].astype(jnp.int32)
    dst = ei[1].astype(jnp.int32)
    ne = src.shape[0]
    ept = -(-ne // (16 * 4 * CHUNK)) * 4 * CHUNK
    pad = 16 * ept - ne
    iota = jnp.arange(pad, dtype=jnp.int32)
    src = jnp.concatenate([src, iota % N_PLAYER])
    dst = jnp.concatenate([dst, N_PLAYER + (iota % (NROWS - N_PLAYER))])
    return src.reshape(16, ept // CHUNK, CHUNK), dst.reshape(16, ept // CHUNK, CHUNK)


_BLK = 1000


def _enc_body(x_ref, w_ref, b_ref, o_ref):
    o_ref[...] = _dotbf(x_ref[...], w_ref[...]) + b_ref[...]


def _encode(x, w_aug, b_aug):
    m, k = x.shape
    w = w_aug.shape[1]
    return pl.pallas_call(
        _enc_body,
        grid=(m // _BLK,),
        in_specs=[
            pl.BlockSpec((_BLK, k), lambda i: (i, 0)),
            pl.BlockSpec((k, w), lambda i: (0, 0)),
            pl.BlockSpec((1, w), lambda i: (0, 0)),
        ],
        out_specs=pl.BlockSpec((_BLK, w), lambda i: (i, 0)),
        out_shape=jax.ShapeDtypeStruct((m, w), jnp.float32),
    )(x, w_aug, b_aug)


def _comb_body(sf_ref, cf_ref, sb_ref, cb_ref, h_ref, wlf_ref, blf_ref,
               wrf_ref, wlb_ref, blb_ref, wrb_ref, o_ref, ps_ref):
    af = sf_ref[...] / jnp.maximum(cf_ref[...], 1.0)
    ab = sb_ref[...] / jnp.maximum(cb_ref[...], 1.0)
    h = h_ref[...]
    out_f = _dotbf(af, wlf_ref[...]) + blf_ref[...] + _dotbf(h, wrf_ref[...])
    out_b = _dotbf(ab, wlb_ref[...]) + blb_ref[...] + _dotbf(h, wrb_ref[...])
    hn = jnp.maximum((out_f + out_b) * 0.5, 0.0)
    o_ref[...] = hn

    @pl.when(pl.program_id(0) == 0)
    def _():
        ps_ref[...] = jnp.zeros_like(ps_ref)

    ps_ref[...] += jnp.sum(hn, axis=0, keepdims=True)


def _combine(sf, cf, sb, cb, h, wlf, blf, wrf, wlb, blb, wrb):
    m = h.shape[0]
    return pl.pallas_call(
        _comb_body,
        grid=(m // _BLK,),
        in_specs=[
            pl.BlockSpec((_BLK, HID), lambda i: (i, 0)),
            pl.BlockSpec((_BLK, 1), lambda i: (i, 0)),
            pl.BlockSpec((_BLK, HID), lambda i: (i, 0)),
            pl.BlockSpec((_BLK, 1), lambda i: (i, 0)),
            pl.BlockSpec((_BLK, HID), lambda i: (i, 0)),
            pl.BlockSpec((HID, HID), lambda i: (0, 0)),
            pl.BlockSpec((1, HID), lambda i: (0, 0)),
            pl.BlockSpec((HID, HID), lambda i: (0, 0)),
            pl.BlockSpec((HID, HID), lambda i: (0, 0)),
            pl.BlockSpec((1, HID), lambda i: (0, 0)),
            pl.BlockSpec((HID, HID), lambda i: (0, 0)),
        ],
        out_specs=[
            pl.BlockSpec((_BLK, HID), lambda i: (i, 0)),
            pl.BlockSpec((1, HID), lambda i: (0, 0)),
        ],
        out_shape=[
            jax.ShapeDtypeStruct((m, HID), jnp.float32),
            jax.ShapeDtypeStruct((1, HID), jnp.float32),
        ],
    )(sf, cf, sb, cb, h, wlf, blf, wrf, wlb, blb, wrb)


def _dot(a, b):
    return _dotbf(a, b)


def _tail_body(ps_ref, wf_ref, vf_ref, role_ref, bat_ref, bowl_ref, exp_ref,
               ww_ref, bw_ref, wve_ref, bve_ref, rolet_ref, batt_ref, bowlt_ref,
               ew_ref, eb_ref, aw_ref, ab_ref, pw_ref, pb_ref, wv_ref, bv_ref,
               wo_ref, bo_ref, w1_ref, b1_ref, w2_ref, b2_ref, w3_ref, b3_ref,
               wa_ref, ba_ref, wb2_ref, bb2_ref,
               mp_ref, wip_ref, pe_ref, att_ref, ve_ref, te_ref):
    pe = ps_ref[...] / float(N_PLAYER)
    weather_emb = _dot(wf_ref[...], ww_ref[...]) + bw_ref[...]
    ve = _dot(vf_ref[...], wve_ref[...]) + bve_ref[...]
    v = _dot(weather_emb, wv_ref[...]) + bv_ref[...]
    att = _dot(v, wo_ref[...]) + bo_ref[...]

    def onehot(idx_ref, depth):
        idx = idx_ref[...]
        io = lax.broadcasted_iota(jnp.int32, (22, depth), 1)
        return (io == idx).astype(jnp.float32)

    def exact_dot(a, b):
        return jnp.dot(a, b, preferred_element_type=jnp.float32, precision=_PREC)

    role_e = exact_dot(onehot(role_ref, 5), rolet_ref[...])
    bat_e = exact_dot(onehot(bat_ref, 3), batt_ref[...])
    bowl_e = exact_dot(onehot(bowl_ref, 9), bowlt_ref[...])
    exp_e = _dot(exp_ref[...], ew_ref[...]) + eb_ref[...]
    pemb = jnp.concatenate([role_e, bat_e, bowl_e, exp_e], axis=1)
    sm = jnp.concatenate(
        [jnp.mean(pemb[0:11, :], axis=0, keepdims=True),
         jnp.mean(pemb[11:22, :], axis=0, keepdims=True)], axis=0)
    squad_emb = _dot(sm, aw_ref[...]) + ab_ref[...]
    te = jnp.mean(_dot(squad_emb, pw_ref[...]) + pb_ref[...], axis=0,
                  keepdims=True)

    combined = jnp.concatenate([pe, att, ve, te], axis=1)
    hh = jnp.maximum(exact_dot(combined, w1_ref[...]) + b1_ref[...], 0.0)
    hh = jnp.maximum(exact_dot(hh, w2_ref[...]) + b2_ref[...], 0.0)
    mp_ref[...] = exact_dot(hh, w3_ref[...]) + b3_ref[...]
    wip_ref[...] = _dot(jnp.maximum(_dot(att, wa_ref[...]) + ba_ref[...], 0.0),
                        wb2_ref[...]) + bb2_ref[...]
    pe_ref[...] = pe
    att_ref[...] = att
    ve_ref[...] = ve
    te_ref[...] = te


def _tail(*args):
    return pl.pallas_call(
        _tail_body,
        out_shape=[
            jax.ShapeDtypeStruct((1, 1), jnp.float32),
            jax.ShapeDtypeStruct((1, 3), jnp.float32),
            jax.ShapeDtypeStruct((1, HID), jnp.float32),
            jax.ShapeDtypeStruct((1, HID), jnp.float32),
            jax.ShapeDtypeStruct((1, HID), jnp.float32),
            jax.ShapeDtypeStruct((1, HID), jnp.float32),
        ],
    )(*args)


def kernel(x_player, x_venue, x_team, x_match, x_weather, ei_faced,
           ei_bowled_to, ei_played_at_pv, ei_plays_for, ei_played_at_mv,
           ei_had_weather, ei_played_in, weather_features, venue_features,
           role_idx, bat_idx, bowl_idx, exp_feats, params):
    sF, dF = _prep_edges(ei_faced)
    sB, dB = _prep_edges(ei_bowled_to)
    src_idx = jnp.stack([sF, sB])
    dst_idx = jnp.stack([dF, dB])
    ch = src_idx.shape[2]

    We, be = params["enc"]["player"]
    W_aug = jnp.pad(We, ((0, 0), (0, 16)))
    b_aug = jnp.concatenate(
        [be, jnp.ones((1,), jnp.float32),
         jnp.zeros((15,), jnp.float32)])[None]
    h_aug = _encode(x_player, W_aug, b_aug)

    seg80 = _make_seg_sum(HID + 16, ch)
    seg64 = _make_seg_sum4(HID, ch)

    acc0 = seg80(src_idx, dst_idx, h_aug)
    cF = acc0[0, :N_PLAYER, HID:HID + 1]
    cB = acc0[1, :N_PLAYER, HID:HID + 1]

    h = h_aug[:, :HID]
    acc = acc0
    ps = None
    for li, layer in enumerate(params["convs"]):
        (WlF, blF, WrF), (WlB, blB, WrB) = layer[0], layer[1]
        h, ps = _combine(
            acc[0, :N_PLAYER, :HID], cF, acc[1, :N_PLAYER, :HID], cB, h,
            WlF, blF[None], WrF, WlB, blB[None], WrB)
        if li < 2:
            acc = seg64(src_idx, dst_idx, h)

    sq = params["squad"]
    at = params["attn"]
    (W1, b1), (W2, b2), (W3, b3) = params["mp"]
    (Wa, ba), (Wb2, bb2) = params["wip"]
    mp, wip, pe, att, ve, te = _tail(
        ps, weather_features[None], venue_features[None],
        role_idx.reshape(22, 1), bat_idx.reshape(22, 1),
        bowl_idx.reshape(22, 1), exp_feats.reshape(22, 4),
        params["weather_enc"][0], params["weather_enc"][1][None],
        params["venue_enc"][0], params["venue_enc"][1][None],
        sq["role"], sq["bat"], sq["bowl"], sq["exp"][0], sq["exp"][1][None],
        sq["agg"][0], sq["agg"][1][None],
        params["proj"][0], params["proj"][1][None],
        at["Wv"][0], at["Wv"][1][None], at["Wo"][0], at["Wo"][1][None],
        W1, b1[None], W2, b2[None], W3, b3[None],
        Wa, ba[None], Wb2, bb2[None])

    return (mp.reshape(1), wip.reshape(3), pe.reshape(HID), att.reshape(HID),
            ve.reshape(HID), te.reshape(HID))
